# initial kernel scaffold (unmeasured)
import jax
import jax.numpy as jnp
from jax import lax
from jax.experimental import pallas as pl
from jax.experimental.pallas import tpu as pltpu

N_DEV = 4
M, N = 4096, 8192


def _exchange_body(part_ref, allbuf_ref, send_sems, recv_sems):
    me = lax.axis_index("i")

    sends = []
    for k in range(1, N_DEV):
        t = (me + k) % N_DEV
        slot_on_t = jnp.where(me > t, me - 1, me)
        rdma = pltpu.make_async_remote_copy(
            src_ref=part_ref,
            dst_ref=allbuf_ref.at[slot_on_t],
            send_sem=send_sems.at[k - 1],
            recv_sem=recv_sems.at[slot_on_t],
            device_id=(t,),
            device_id_type=pl.DeviceIdType.MESH,
        )
        rdma.start()
        sends.append(rdma)

    for k in range(1, N_DEV):
        s = (me - k) % N_DEV
        slot = jnp.where(s > me, s - 1, s)
        recv = pltpu.make_async_remote_copy(
            src_ref=part_ref,
            dst_ref=allbuf_ref.at[slot],
            send_sem=send_sems.at[0],
            recv_sem=recv_sems.at[slot],
            device_id=(s,),
            device_id_type=pl.DeviceIdType.MESH,
        )
        recv.wait_recv()

    for rdma in sends:
        rdma.wait_send()


def kernel(x, w_mat, scale_x, scale_w):
    part = lax.dot_general(
        x.astype(jnp.bfloat16),
        w_mat.astype(jnp.bfloat16),
        dimension_numbers=(((1,), (0,)), ((), ())),
        preferred_element_type=jnp.float32,
    )

    gathered = pl.pallas_call(
        _exchange_body,
        out_shape=jax.ShapeDtypeStruct((N_DEV - 1, M, N), jnp.float32),
        in_specs=[pl.BlockSpec(memory_space=pltpu.ANY)],
        out_specs=pl.BlockSpec(memory_space=pltpu.ANY),
        scratch_shapes=[
            pltpu.SemaphoreType.DMA((N_DEV - 1,)),
            pltpu.SemaphoreType.DMA((N_DEV - 1,)),
        ],
        compiler_params=pltpu.CompilerParams(collective_id=0),
    )(part)

    total = part + gathered[0] + gathered[1] + gathered[2]
    y = total * (scale_x[0] * scale_w[0])
    return y * jax.nn.sigmoid(jnp.clip(y, -60.0, 60.0))


# baseline (device time: 3199945 ns/iter reference)
import jax
import jax.numpy as jnp
from jax import lax
from jax.experimental import pallas as pl
from jax.experimental.pallas import tpu as pltpu

N_DEV = 4
M, N = 4096, 8192


def _exchange_body(part_ref, allbuf_ref, send_sems, recv_sems):
    me = lax.axis_index("i")

    sends = []
    for k in range(1, N_DEV):
        t = (me + k) % N_DEV
        slot_on_t = jnp.where(me > t, me - 1, me)
        rdma = pltpu.make_async_remote_copy(
            src_ref=part_ref,
            dst_ref=allbuf_ref.at[slot_on_t],
            send_sem=send_sems.at[k - 1],
            recv_sem=recv_sems.at[slot_on_t],
            device_id=(t,),
            device_id_type=pl.DeviceIdType.MESH,
        )
        rdma.start()
        sends.append(rdma)

    for k in range(1, N_DEV):
        s = (me - k) % N_DEV
        slot = jnp.where(s > me, s - 1, s)
        recv = pltpu.make_async_remote_copy(
            src_ref=part_ref,
            dst_ref=allbuf_ref.at[slot],
            send_sem=send_sems.at[0],
            recv_sem=recv_sems.at[slot],
            device_id=(s,),
            device_id_type=pl.DeviceIdType.MESH,
        )
        recv.wait_recv()

    for rdma in sends:
        rdma.wait_send()


def kernel(x, w_mat, scale_x, scale_w):
    part = lax.dot_general(
        x.astype(jnp.bfloat16),
        w_mat.astype(jnp.bfloat16),
        dimension_numbers=(((1,), (0,)), ((), ())),
        preferred_element_type=jnp.float32,
    )

    gathered = pl.pallas_call(
        _exchange_body,
        out_shape=jax.ShapeDtypeStruct((N_DEV - 1, M, N), jnp.float32),
        in_specs=[pl.BlockSpec(memory_space=pl.ANY)],
        out_specs=pl.BlockSpec(memory_space=pl.ANY),
        scratch_shapes=[
            pltpu.SemaphoreType.DMA((N_DEV - 1,)),
            pltpu.SemaphoreType.DMA((N_DEV - 1,)),
        ],
    )(part)

    total = part + gathered[0] + gathered[1] + gathered[2]
    y = total * (scale_x[0] * scale_w[0])
    return y * jax.nn.sigmoid(jnp.clip(y, -60.0, 60.0))


# device time: 666475 ns/iter; 4.8013x vs baseline; 4.8013x over previous
import jax
import jax.numpy as jnp
from jax import lax
from jax.experimental import pallas as pl
from jax.experimental.pallas import tpu as pltpu

N_DEV = 4
M, K, N = 4096, 4096, 8192
KS = K // N_DEV

BM, BN = 1024, 2048


def _ag_body(x_ref, w_ref, gx_ref, gw_ref, local_sems, send_sems, recv_sems):
    me = lax.axis_index("i")
    right = (me + 1) % N_DEV
    left = (me - 1) % N_DEV

    cp_x = pltpu.make_async_copy(x_ref, gx_ref.at[me], local_sems.at[0])
    cp_w = pltpu.make_async_copy(w_ref, gw_ref.at[me], local_sems.at[1])
    cp_x.start()
    cp_w.start()
    cp_x.wait()
    cp_w.wait()

    for h in range(N_DEV - 1):
        s = (me - h) % N_DEV
        r = (me - h - 1) % N_DEV
        sends = []
        for t, (buf, sems, rsems) in enumerate(
            [(gx_ref, send_sems, recv_sems), (gw_ref, send_sems, recv_sems)]
        ):
            rdma = pltpu.make_async_remote_copy(
                src_ref=buf.at[s],
                dst_ref=buf.at[s],
                send_sem=sems.at[t, h],
                recv_sem=rsems.at[t, h],
                device_id=(right,),
                device_id_type=pl.DeviceIdType.MESH,
            )
            rdma.start()
            sends.append(rdma)
        for t, buf in enumerate([gx_ref, gw_ref]):
            recv = pltpu.make_async_remote_copy(
                src_ref=buf.at[s],
                dst_ref=buf.at[r],
                send_sem=send_sems.at[t, h],
                recv_sem=recv_sems.at[t, h],
                device_id=(left,),
                device_id_type=pl.DeviceIdType.MESH,
            )
            recv.wait_recv()
        for rdma in sends:
            rdma.wait_send()


def _gemm_body(scale_ref, gx_ref, gw_ref, o_ref, acc_ref):
    k = pl.program_id(2)

    @pl.when(k == 0)
    def _():
        acc_ref[...] = jnp.zeros_like(acc_ref)

    acc_ref[...] += jnp.dot(
        gx_ref[0], gw_ref[0], preferred_element_type=jnp.float32
    )

    @pl.when(k == N_DEV - 1)
    def _():
        y = acc_ref[...] * scale_ref[0]
        o_ref[...] = y * jax.nn.sigmoid(y)


def kernel(x, w_mat, scale_x, scale_w):
    x8 = x.astype(jnp.float8_e4m3fn)
    w8 = w_mat.astype(jnp.float8_e5m2)

    gx, gw = pl.pallas_call(
        _ag_body,
        out_shape=(
            jax.ShapeDtypeStruct((N_DEV, M, KS), jnp.float8_e4m3fn),
            jax.ShapeDtypeStruct((N_DEV, KS, N), jnp.float8_e5m2),
        ),
        in_specs=[
            pl.BlockSpec(memory_space=pl.ANY),
            pl.BlockSpec(memory_space=pl.ANY),
        ],
        out_specs=(
            pl.BlockSpec(memory_space=pl.ANY),
            pl.BlockSpec(memory_space=pl.ANY),
        ),
        scratch_shapes=[
            pltpu.SemaphoreType.DMA((2,)),
            pltpu.SemaphoreType.DMA((2, N_DEV - 1)),
            pltpu.SemaphoreType.DMA((2, N_DEV - 1)),
        ],
    )(x8, w8)

    scale = (scale_x * scale_w).astype(jnp.float32)

    return pl.pallas_call(
        _gemm_body,
        grid=(M // BM, N // BN, N_DEV),
        in_specs=[
            pl.BlockSpec(memory_space=pltpu.SMEM),
            pl.BlockSpec((1, BM, KS), lambda m, n, k: (k, m, 0)),
            pl.BlockSpec((1, KS, BN), lambda m, n, k: (k, 0, n)),
        ],
        out_specs=pl.BlockSpec((BM, BN), lambda m, n, k: (m, n)),
        out_shape=jax.ShapeDtypeStruct((M, N), jnp.float32),
        scratch_shapes=[pltpu.VMEM((BM, BN), jnp.float32)],
        compiler_params=pltpu.CompilerParams(
            dimension_semantics=("parallel", "parallel", "arbitrary"),
        ),
    )(scale, gx, gw)


# device time: 458282 ns/iter; 6.9825x vs baseline; 1.4543x over previous
import jax
import jax.numpy as jnp
from jax import lax
from jax.experimental import pallas as pl
from jax.experimental.pallas import tpu as pltpu

N_DEV = 4
M, K, N = 4096, 4096, 8192
KS = K // N_DEV

BM, BN = 1024, 2048


MH, KH = M // 2, KS // 2


def _ag_body(x_ref, w_ref, gx_ref, gw_ref, local_sems, send_sems, recv_sems):
    me = lax.axis_index("i")
    right = (me + 1) % N_DEV
    left = (me - 1) % N_DEV

    cp_x = pltpu.make_async_copy(x_ref, gx_ref.at[me], local_sems.at[0])
    cp_w = pltpu.make_async_copy(w_ref, gw_ref.at[me], local_sems.at[1])
    cp_x.start()
    cp_w.start()

    def halves(h):
        sR = (me - h) % N_DEV
        sL = (me + h) % N_DEV
        srcs = [x_ref, w_ref] if h == 0 else None
        out = []
        for t, (buf, hh) in enumerate([(gx_ref, MH), (gw_ref, KH)]):
            src_t = srcs[t] if srcs is not None else None
            out.append((t, 0, buf, hh, sR, right, 0, src_t))
            out.append((t, 1, buf, hh, sL, left, hh, src_t))
        return out

    for h in range(N_DEV - 1):
        sends = []
        for t, d, buf, hh, s, tgt, off, src_t in halves(h):
            src = (
                src_t.at[pl.ds(off, hh)]
                if src_t is not None
                else buf.at[s, pl.ds(off, hh)]
            )
            rdma = pltpu.make_async_remote_copy(
                src_ref=src,
                dst_ref=buf.at[s, pl.ds(off, hh)],
                send_sem=send_sems.at[t, d, h],
                recv_sem=recv_sems.at[t, d, h],
                device_id=(tgt,),
                device_id_type=pl.DeviceIdType.MESH,
            )
            rdma.start()
            sends.append(rdma)
        for t, (buf, hh) in enumerate([(gx_ref, MH), (gw_ref, KH)]):
            for d, r in [(0, (me - h - 1) % N_DEV), (1, (me + h + 1) % N_DEV)]:
                off = 0 if d == 0 else hh
                recv = pltpu.make_async_remote_copy(
                    src_ref=buf.at[r, pl.ds(off, hh)],
                    dst_ref=buf.at[r, pl.ds(off, hh)],
                    send_sem=send_sems.at[t, d, h],
                    recv_sem=recv_sems.at[t, d, h],
                    device_id=(left,),
                    device_id_type=pl.DeviceIdType.MESH,
                )
                recv.wait_recv()
        for rdma in sends:
            rdma.wait_send()

    cp_x.wait()
    cp_w.wait()


def _gemm_body(scale_ref, gx_ref, gw_ref, o_ref, acc_ref):
    k = pl.program_id(2)

    @pl.when(k == 0)
    def _():
        acc_ref[...] = jnp.zeros_like(acc_ref)

    acc_ref[...] += jnp.dot(
        gx_ref[0], gw_ref[0], preferred_element_type=jnp.float32
    )

    @pl.when(k == N_DEV - 1)
    def _():
        y = acc_ref[...] * scale_ref[0]
        o_ref[...] = y * jax.nn.sigmoid(y)


def kernel(x, w_mat, scale_x, scale_w):
    x8 = x.astype(jnp.float8_e4m3fn)
    w8 = w_mat.astype(jnp.float8_e5m2)

    gx, gw = pl.pallas_call(
        _ag_body,
        out_shape=(
            jax.ShapeDtypeStruct((N_DEV, M, KS), jnp.float8_e4m3fn),
            jax.ShapeDtypeStruct((N_DEV, KS, N), jnp.float8_e5m2),
        ),
        in_specs=[
            pl.BlockSpec(memory_space=pl.ANY),
            pl.BlockSpec(memory_space=pl.ANY),
        ],
        out_specs=(
            pl.BlockSpec(memory_space=pl.ANY),
            pl.BlockSpec(memory_space=pl.ANY),
        ),
        scratch_shapes=[
            pltpu.SemaphoreType.DMA((2,)),
            pltpu.SemaphoreType.DMA((2, 2, N_DEV - 1)),
            pltpu.SemaphoreType.DMA((2, 2, N_DEV - 1)),
        ],
    )(x8, w8)

    scale = (scale_x * scale_w).astype(jnp.float32)

    return pl.pallas_call(
        _gemm_body,
        grid=(M // BM, N // BN, N_DEV),
        in_specs=[
            pl.BlockSpec(memory_space=pltpu.SMEM),
            pl.BlockSpec((1, BM, KS), lambda m, n, k: (k, m, 0)),
            pl.BlockSpec((1, KS, BN), lambda m, n, k: (k, 0, n)),
        ],
        out_specs=pl.BlockSpec((BM, BN), lambda m, n, k: (m, n)),
        out_shape=jax.ShapeDtypeStruct((M, N), jnp.float32),
        scratch_shapes=[pltpu.VMEM((BM, BN), jnp.float32)],
        compiler_params=pltpu.CompilerParams(
            dimension_semantics=("parallel", "parallel", "arbitrary"),
        ),
    )(scale, gx, gw)


# device time: 414256 ns/iter; 7.7246x vs baseline; 1.1063x over previous
import jax
import jax.numpy as jnp
from jax import lax
from jax.experimental import pallas as pl
from jax.experimental.pallas import tpu as pltpu

N_DEV = 4
M, K, N = 4096, 4096, 8192
KS = K // N_DEV

BM, BN = 1024, 2048


MH, KH = M // 2, KS // 2


def _ag_body(x_ref, w_ref, gx_ref, gw_ref, local_sems, send_sems, recv_sems):
    me = lax.axis_index("i")
    right = (me + 1) % N_DEV
    left = (me - 1) % N_DEV

    cp_x = pltpu.make_async_copy(x_ref, gx_ref.at[me], local_sems.at[0])
    cp_w = pltpu.make_async_copy(w_ref, gw_ref.at[me], local_sems.at[1])
    cp_x.start()
    cp_w.start()

    def halves(h):
        sR = (me - h) % N_DEV
        sL = (me + h) % N_DEV
        srcs = [x_ref, w_ref] if h == 0 else None
        out = []
        for t, (buf, hh) in enumerate([(gx_ref, MH), (gw_ref, KH)]):
            src_t = srcs[t] if srcs is not None else None
            out.append((t, 0, buf, hh, sR, right, 0, src_t))
            out.append((t, 1, buf, hh, sL, left, hh, src_t))
        return out

    for h in range(N_DEV - 1):
        sends = []
        for t, d, buf, hh, s, tgt, off, src_t in halves(h):
            src = (
                src_t.at[pl.ds(off, hh)]
                if src_t is not None
                else buf.at[s, pl.ds(off, hh)]
            )
            rdma = pltpu.make_async_remote_copy(
                src_ref=src,
                dst_ref=buf.at[s, pl.ds(off, hh)],
                send_sem=send_sems.at[t, d, h],
                recv_sem=recv_sems.at[t, d, h],
                device_id=(tgt,),
                device_id_type=pl.DeviceIdType.MESH,
            )
            rdma.start()
            sends.append(rdma)
        for t, (buf, hh) in enumerate([(gx_ref, MH), (gw_ref, KH)]):
            for d, r in [(0, (me - h - 1) % N_DEV), (1, (me + h + 1) % N_DEV)]:
                off = 0 if d == 0 else hh
                recv = pltpu.make_async_remote_copy(
                    src_ref=buf.at[r, pl.ds(off, hh)],
                    dst_ref=buf.at[r, pl.ds(off, hh)],
                    send_sem=send_sems.at[t, d, h],
                    recv_sem=recv_sems.at[t, d, h],
                    device_id=(left,),
                    device_id_type=pl.DeviceIdType.MESH,
                )
                recv.wait_recv()
        for rdma in sends:
            rdma.wait_send()

    cp_x.wait()
    cp_w.wait()


def _gemm_body(scale_ref, gx_ref, gw_ref, o_ref, acc_ref):
    m = pl.program_id(1)
    acc_ref[...] = jnp.dot(
        gx_ref[0, pl.ds(m * BM, BM), :],
        gw_ref[0],
        preferred_element_type=jnp.float32,
    )
    for k in range(1, N_DEV):
        acc_ref[...] += jnp.dot(
            gx_ref[k, pl.ds(m * BM, BM), :],
            gw_ref[k],
            preferred_element_type=jnp.float32,
        )
    y = acc_ref[...] * scale_ref[0]
    o_ref[...] = y * jax.nn.sigmoid(y)


def kernel(x, w_mat, scale_x, scale_w):
    x8 = x.astype(jnp.float8_e4m3fn)
    w8 = w_mat.astype(jnp.float8_e5m2)

    gx, gw = pl.pallas_call(
        _ag_body,
        out_shape=(
            jax.ShapeDtypeStruct((N_DEV, M, KS), jnp.float8_e4m3fn),
            jax.ShapeDtypeStruct((N_DEV, KS, N), jnp.float8_e5m2),
        ),
        in_specs=[
            pl.BlockSpec(memory_space=pl.ANY),
            pl.BlockSpec(memory_space=pl.ANY),
        ],
        out_specs=(
            pl.BlockSpec(memory_space=pl.ANY),
            pl.BlockSpec(memory_space=pl.ANY),
        ),
        scratch_shapes=[
            pltpu.SemaphoreType.DMA((2,)),
            pltpu.SemaphoreType.DMA((2, 2, N_DEV - 1)),
            pltpu.SemaphoreType.DMA((2, 2, N_DEV - 1)),
        ],
    )(x8, w8)

    scale = (scale_x * scale_w).astype(jnp.float32)

    return pl.pallas_call(
        _gemm_body,
        grid=(N // BN, M // BM),
        in_specs=[
            pl.BlockSpec(memory_space=pltpu.SMEM),
            pl.BlockSpec(memory_space=pltpu.VMEM),
            pl.BlockSpec((N_DEV, KS, BN), lambda n, m: (0, 0, n)),
        ],
        out_specs=pl.BlockSpec((BM, BN), lambda n, m: (m, n)),
        out_shape=jax.ShapeDtypeStruct((M, N), jnp.float32),
        scratch_shapes=[pltpu.VMEM((BM, BN), jnp.float32)],
        compiler_params=pltpu.CompilerParams(
            dimension_semantics=("parallel", "parallel"),
            vmem_limit_bytes=100 * 1024 * 1024,
        ),
    )(scale, gx, gw)


# device time: 410308 ns/iter; 7.7989x vs baseline; 1.0096x over previous
import jax
import jax.numpy as jnp
from jax import lax
from jax.experimental import pallas as pl
from jax.experimental.pallas import tpu as pltpu

N_DEV = 4
M, K, N = 4096, 4096, 8192
KS = K // N_DEV

BM, BN = 1024, 2048
MH, KH = M // 2, KS // 2


def _ag_body(x_ref, w_ref, gx_ref, gw_ref, local_sems, send_sems, recv_sems):
    me = lax.axis_index("i")
    right = (me + 1) % N_DEV
    left = (me - 1) % N_DEV

    cp_x = pltpu.make_async_copy(
        x_ref, gx_ref.at[:, pl.ds(me * KS, KS)], local_sems.at[0]
    )
    cp_w = pltpu.make_async_copy(
        w_ref, gw_ref.at[pl.ds(me * KS, KS)], local_sems.at[1]
    )
    cp_x.start()
    cp_w.start()

    def x_slice(s, d):
        return gx_ref.at[pl.ds(0 if d == 0 else MH, MH), pl.ds(s * KS, KS)]

    def w_slice(s, d):
        return gw_ref.at[pl.ds(s * KS + (0 if d == 0 else KH), KH)]

    for h in range(N_DEV - 1):
        sR = (me - h) % N_DEV
        sL = (me + h) % N_DEV
        sends = []
        for t, slc in enumerate([x_slice, w_slice]):
            for d, s, tgt in [(0, sR, right), (1, sL, left)]:
                if h == 0:
                    inp, hh = (x_ref, MH) if t == 0 else (w_ref, KH)
                    src = inp.at[pl.ds(0 if d == 0 else hh, hh)]
                else:
                    src = slc(s, d)
                rdma = pltpu.make_async_remote_copy(
                    src_ref=src,
                    dst_ref=slc(s, d),
                    send_sem=send_sems.at[t, d, h],
                    recv_sem=recv_sems.at[t, d, h],
                    device_id=(tgt,),
                    device_id_type=pl.DeviceIdType.MESH,
                )
                rdma.start()
                sends.append(rdma)
        for t, slc in enumerate([x_slice, w_slice]):
            for d, r in [(0, (me - h - 1) % N_DEV), (1, (me + h + 1) % N_DEV)]:
                recv = pltpu.make_async_remote_copy(
                    src_ref=slc(r, d),
                    dst_ref=slc(r, d),
                    send_sem=send_sems.at[t, d, h],
                    recv_sem=recv_sems.at[t, d, h],
                    device_id=(left,),
                    device_id_type=pl.DeviceIdType.MESH,
                )
                recv.wait_recv()
        for rdma in sends:
            rdma.wait_send()

    cp_x.wait()
    cp_w.wait()


def _gemm_body(scale_ref, gx_ref, gw_ref, o_ref):
    m = pl.program_id(1)
    y = jnp.dot(
        gx_ref[pl.ds(m * BM, BM), :],
        gw_ref[...],
        preferred_element_type=jnp.float32,
    ) * scale_ref[0]
    o_ref[...] = y * jax.nn.sigmoid(y)


def kernel(x, w_mat, scale_x, scale_w):
    x8 = x.astype(jnp.float8_e4m3fn)
    w8 = w_mat.astype(jnp.float8_e5m2)

    gx, gw = pl.pallas_call(
        _ag_body,
        out_shape=(
            jax.ShapeDtypeStruct((M, K), jnp.float8_e4m3fn),
            jax.ShapeDtypeStruct((K, N), jnp.float8_e5m2),
        ),
        in_specs=[
            pl.BlockSpec(memory_space=pl.ANY),
            pl.BlockSpec(memory_space=pl.ANY),
        ],
        out_specs=(
            pl.BlockSpec(memory_space=pl.ANY),
            pl.BlockSpec(memory_space=pl.ANY),
        ),
        scratch_shapes=[
            pltpu.SemaphoreType.DMA((2,)),
            pltpu.SemaphoreType.DMA((2, 2, N_DEV - 1)),
            pltpu.SemaphoreType.DMA((2, 2, N_DEV - 1)),
        ],
    )(x8, w8)

    scale = (scale_x * scale_w).astype(jnp.float32)

    return pl.pallas_call(
        _gemm_body,
        grid=(N // BN, M // BM),
        in_specs=[
            pl.BlockSpec(memory_space=pltpu.SMEM),
            pl.BlockSpec(memory_space=pltpu.VMEM),
            pl.BlockSpec((K, BN), lambda n, m: (0, n)),
        ],
        out_specs=pl.BlockSpec((BM, BN), lambda n, m: (m, n)),
        out_shape=jax.ShapeDtypeStruct((M, N), jnp.float32),
        compiler_params=pltpu.CompilerParams(
            dimension_semantics=("parallel", "parallel"),
            vmem_limit_bytes=100 * 1024 * 1024,
        ),
    )(scale, gx, gw)
